# baseline (speedup 1.0000x reference)
import functools

import jax
import jax.numpy as jnp
from jax import lax
from jax.experimental import pallas as pl
from jax.experimental.pallas import tpu as pltpu
from jax.experimental.pallas import tpu_sc as plsc

NW = 32
LANES = 16


def kernel(x, W_E):
    B0, B1 = x.shape
    D = W_E.shape[1]
    IB = B0 // 128
    FB = D // 8

    xr = x.reshape(IB, 128, B1).transpose(0, 2, 1)

    mesh = plsc.VectorSubcoreMesh(core_axis_name="c", subcore_axis_name="s")

    @functools.partial(
        pl.kernel,
        mesh=mesh,
        out_type=jax.ShapeDtypeStruct((B1, FB, IB, 8, 128), jnp.float32),
        scratch_types=[
            pltpu.VMEM((B1, 128), jnp.int32),
            pltpu.VMEM((128, D), jnp.float32),
            pltpu.VMEM((FB, 8, 128), jnp.float32),
            pltpu.SemaphoreType.DMA,
        ],
        compiler_params=pltpu.CompilerParams(
            use_tc_tiling_on_sc=False, needs_layout_passes=False),
    )
    def emb(xr_hbm, w_hbm, out_hbm, xr_v, rows_v, t_v, gsem):
        w = lax.axis_index("s") * 2 + lax.axis_index("c")
        pltpu.sync_copy(xr_hbm.at[w], xr_v)
        lane = lax.broadcasted_iota(jnp.int32, (LANES,), 0)

        def body(j, carry):
            pltpu.async_copy(w_hbm.at[xr_v.at[j]], rows_v, gsem).wait()
            for fb in range(FB):
                for fi in range(8):
                    col = jnp.full((LANES,), 8 * fb + fi, jnp.int32)
                    for g in range(128 // LANES):
                        v = plsc.load_gather(
                            rows_v, [lane + LANES * g, col])
                        t_v[fb, fi, pl.ds(LANES * g, LANES)] = v
                pltpu.sync_copy(t_v.at[fb], out_hbm.at[j, fb, w])
            return carry

        lax.fori_loop(0, B1, body, 0)

    out5 = emb(xr, W_E)
    return out5.transpose(2, 4, 0, 1, 3).reshape(B0, B1, D)


# speedup vs baseline: 1.4359x; 1.4359x over previous
import functools

import jax
import jax.numpy as jnp
from jax import lax
from jax.experimental import pallas as pl
from jax.experimental.pallas import tpu as pltpu
from jax.experimental.pallas import tpu_sc as plsc

NW = 32
JCH = 8


def kernel(x, W_E):
    B0, B1 = x.shape
    D = W_E.shape[1]
    IB = B0 // 128
    n_ch = B1 // JCH
    CH = JCH * 128

    xr = x.reshape(IB, 128, B1).transpose(0, 2, 1)

    mesh = plsc.VectorSubcoreMesh(core_axis_name="c", subcore_axis_name="s")

    @functools.partial(
        pl.kernel,
        mesh=mesh,
        out_type=jax.ShapeDtypeStruct((B1, B0, D), jnp.float32),
        scratch_types=[
            pltpu.VMEM((B1 * 128,), jnp.int32),
            pltpu.VMEM((CH, D), jnp.float32),
            pltpu.VMEM((CH, D), jnp.float32),
            pltpu.SemaphoreType.DMA,
            pltpu.SemaphoreType.DMA,
            pltpu.SemaphoreType.DMA,
            pltpu.SemaphoreType.DMA,
        ],
        compiler_params=pltpu.CompilerParams(
            use_tc_tiling_on_sc=False, needs_layout_passes=False),
    )
    def emb(xr_hbm, w_hbm, un_hbm, idx_v, rows0, rows1, g0, g1, o0, o1):
        w = lax.axis_index("s") * 2 + lax.axis_index("c")
        pltpu.sync_copy(xr_hbm.at[w], idx_v)

        rows = [rows0, rows1]
        gsem = [g0, g1]
        osem = [o0, o1]
        gather = [None, None]
        wback = [[], []]

        gather[0] = pltpu.async_copy(
            w_hbm.at[idx_v.at[pl.ds(0, CH)]], rows[0], gsem[0])
        for c in range(n_ch):
            b = c % 2
            nb = (c + 1) % 2
            if c + 1 < n_ch:
                for h in wback[nb]:
                    h.wait()
                wback[nb] = []
                gather[nb] = pltpu.async_copy(
                    w_hbm.at[idx_v.at[pl.ds((c + 1) * CH, CH)]],
                    rows[nb], gsem[nb])
            gather[b].wait()
            for jj in range(JCH):
                wback[b].append(pltpu.async_copy(
                    rows[b].at[pl.ds(jj * 128, 128)],
                    un_hbm.at[c * JCH + jj, pl.ds(128 * w, 128)],
                    osem[b]))
        for h in wback[0] + wback[1]:
            h.wait()

    un = emb(xr.reshape(IB, B1 * 128), W_E)
    return un.transpose(1, 0, 2)
